# baseline (device time: 70825 ns/iter reference)
import os as _os

import jax
import jax.numpy as jnp
from jax import lax
from jax.experimental import pallas as pl
from jax.experimental.pallas import tpu as pltpu

N_DEV = 8
B, SQ, SKV, D = 4, 256, 1024, 1024
HQ_PER = 8
HKV_PER = 2
DH = 128
SCALE = 0.08838834764831843

ROWS = B * SQ
CHUNK = SQ // N_DEV

_SKIP_COMM = _os.environ.get("KERNEL_SKIP_COMM") == "1"


def _fused_body(x_ref, wq_ref, wo_ref, kext_ref, vext_ref, out_ref,
                comm_ref, kbuf, vbuf, kv_sems,
                rs_send_sems, rs_recv_sems, ag_send_sems, ag_recv_sems):
    my = lax.axis_index("i")

    barrier_sem = pltpu.get_barrier_semaphore()
    for k in range(1, N_DEV):
        pl.semaphore_signal(
            barrier_sem, inc=1,
            device_id=(lax.rem(my + k, N_DEV),),
            device_id_type=pl.DeviceIdType.MESH,
        )
    pl.semaphore_wait(barrier_sem, N_DEV - 1)

    copies = []
    for b in range(B):
        for g in range(HKV_PER):
            h = HKV_PER * my + g
            kc = pltpu.make_async_copy(
                kext_ref.at[b, :, h, :], kbuf.at[b, g],
                kv_sems.at[2 * (b * HKV_PER + g)],
            )
            vc = pltpu.make_async_copy(
                vext_ref.at[b, :, h, :], vbuf.at[b, g],
                kv_sems.at[2 * (b * HKV_PER + g) + 1],
            )
            kc.start()
            vc.start()
            copies.extend((kc, vc))

    Q = jnp.dot(
        x_ref[:, :].astype(jnp.bfloat16), wq_ref[:, :].astype(jnp.bfloat16),
        preferred_element_type=jnp.float32,
    ) * SCALE

    for c in copies:
        c.wait()

    wo_b = wo_ref[:, :].astype(jnp.bfloat16)

    rs_rdmas = [None] * B
    ag_rdmas = [None] * B

    def rs_start(b):
        rds = []
        for k in range(1, N_DEV):
            tgt = lax.rem(my + k, N_DEV)
            rdma = pltpu.make_async_remote_copy(
                src_ref=out_ref.at[pl.ds(SQ * b + CHUNK * tgt, CHUNK), :],
                dst_ref=comm_ref.at[b, k],
                send_sem=rs_send_sems.at[b, k],
                recv_sem=rs_recv_sems.at[b, k],
                device_id=(tgt,),
                device_id_type=pl.DeviceIdType.MESH,
            )
            rdma.start()
            rds.append(rdma)
        rs_rdmas[b] = rds

    def rs_finish_ag_start(b):
        for r in rs_rdmas[b]:
            r.wait()
        mine = pl.ds(SQ * b + CHUNK * my, CHUNK)
        total = comm_ref[b, 1]
        for k in range(2, N_DEV):
            total = total + comm_ref[b, k]
        out_ref[mine, :] += total
        rds = []
        for k in range(1, N_DEV):
            rdma = pltpu.make_async_remote_copy(
                src_ref=out_ref.at[mine, :],
                dst_ref=out_ref.at[mine, :],
                send_sem=ag_send_sems.at[b, k],
                recv_sem=ag_recv_sems.at[b, k],
                device_id=(lax.rem(my + k, N_DEV),),
                device_id_type=pl.DeviceIdType.MESH,
            )
            rdma.start()
            rds.append(rdma)
        ag_rdmas[b] = rds

    for b in range(B):
        os = []
        for g in range(HKV_PER):
            kbg = kbuf[b, g, :, :].astype(jnp.bfloat16)
            vbg = vbuf[b, g, :, :].astype(jnp.bfloat16)
            for r in range(HQ_PER // HKV_PER):
                t = g * (HQ_PER // HKV_PER) + r
                qh = Q[b * SQ:(b + 1) * SQ, t * DH:(t + 1) * DH]
                s = lax.dot_general(
                    qh.astype(jnp.bfloat16), kbg,
                    (((1,), (1,)), ((), ())),
                    preferred_element_type=jnp.float32,
                )
                e = jnp.exp(s)
                l = jnp.sum(e, axis=1, keepdims=True)
                o = jnp.dot(e.astype(jnp.bfloat16), vbg,
                            preferred_element_type=jnp.float32)
                os.append((o / l).astype(jnp.bfloat16))
        attn_b = jnp.concatenate(os, axis=1)
        out_ref[b * SQ:(b + 1) * SQ, :] = jnp.dot(
            attn_b, wo_b, preferred_element_type=jnp.float32)
        if not _SKIP_COMM:
            rs_start(b)
            if b >= 1:
                rs_finish_ag_start(b - 1)

    if not _SKIP_COMM:
        rs_finish_ag_start(B - 1)
        for b in range(B):
            for r in ag_rdmas[b]:
                r.wait()


def kernel(x, Wq, Wo, K_ext, V_ext):
    out = pl.pallas_call(
        _fused_body,
        out_shape=jax.ShapeDtypeStruct((ROWS, D), jnp.float32),
        in_specs=[
            pl.BlockSpec(memory_space=pltpu.VMEM),
            pl.BlockSpec(memory_space=pltpu.VMEM),
            pl.BlockSpec(memory_space=pltpu.VMEM),
            pl.BlockSpec(memory_space=pl.ANY),
            pl.BlockSpec(memory_space=pl.ANY),
        ],
        out_specs=pl.BlockSpec(memory_space=pltpu.VMEM),
        scratch_shapes=[
            pltpu.VMEM((B, N_DEV, CHUNK, D), jnp.float32),
            pltpu.VMEM((B, HKV_PER, SKV, DH), jnp.float32),
            pltpu.VMEM((B, HKV_PER, SKV, DH), jnp.float32),
            pltpu.SemaphoreType.DMA((2 * B * HKV_PER,)),
            pltpu.SemaphoreType.DMA((B, N_DEV)),
            pltpu.SemaphoreType.DMA((B, N_DEV)),
            pltpu.SemaphoreType.DMA((B, N_DEV)),
            pltpu.SemaphoreType.DMA((B, N_DEV)),
        ],
        compiler_params=pltpu.CompilerParams(collective_id=0),
    )(x.reshape(ROWS, D), Wq, Wo, K_ext, V_ext)
    return out.reshape(B, SQ, D)
